# baseline (device time: 19572 ns/iter reference)
import numpy as np
import jax
import jax.numpy as jnp
from jax import lax
from jax.experimental import pallas as pl
from jax.experimental.pallas import tpu as pltpu

N_DEV = 4
B = 2
SQ_LOCAL = 256
SQ = SQ_LOCAL * N_DEV
D = 768
HQ = 4
DH = 64
DM = HQ * DH
SCALE = 1.0 / np.sqrt(DH)

BF16 = jnp.bfloat16
INT8 = jnp.int8

OWN, L, R, OPP = 0, 1, 2, 3
K_, V_ = 0, 1

_TABS = None


def _rope_tables():
    global _TABS
    if _TABS is None:
        inv = 1.0 / (10000.0 ** (np.arange(0, DH, 2) / DH))
        pos = np.arange(SQ)[:, None] * inv[None, :]
        cos = np.repeat(np.cos(pos), 2, axis=-1)
        sin = np.repeat(np.sin(pos), 2, axis=-1)
        _TABS = (jnp.asarray(np.tile(cos, (1, HQ)), dtype=jnp.float32),
                 jnp.asarray(np.tile(sin, (1, HQ)), dtype=jnp.float32))
    return _TABS


def kernel(x, Wq, Wk, Wv, Wo):
    cos_tab, sin_tab = _rope_tables()
    my_out = lax.axis_index("i")
    cos_l = lax.dynamic_slice_in_dim(cos_tab, my_out * SQ_LOCAL, SQ_LOCAL, 0)
    sin_l = lax.dynamic_slice_in_dim(sin_tab, my_out * SQ_LOCAL, SQ_LOCAL, 0)

    def body(x_ref, wq_ref, wk_ref, wv_ref, wo_ref, cos_ref, sin_ref,
             out_ref, commk_ref, commv_ref, commsc_ref, send_sems, recv_sems):
        my = lax.axis_index("i")
        left = lax.rem(my + N_DEV - 1, N_DEV)
        right = lax.rem(my + 1, N_DEV)

        cos_2 = jnp.concatenate([cos_ref[:, :], cos_ref[:, :]], axis=0)
        sin_2 = jnp.concatenate([sin_ref[:, :], sin_ref[:, :]], axis=0)
        lane = lax.broadcasted_iota(jnp.int32, (B * SQ_LOCAL, DM), 1)
        even = (lane % 2) == 0

        def rope(t):
            r = jnp.where(even, -pltpu.roll(t, DM - 1, 1), pltpu.roll(t, 1, 1))
            return t * cos_2 + r * sin_2

        def quant(t):
            cmax = jnp.maximum(
                jnp.max(jnp.abs(t), axis=0, keepdims=True), 1e-20)
            q = jnp.clip(jnp.round(t * (127.0 / cmax)), -127.0, 127.0)
            return q.astype(INT8), cmax * (1.0 / 127.0)

        x2 = x_ref[:, :, :].reshape(B * SQ_LOCAL, D)

        k_q, k_sc = quant(rope(jnp.dot(x2, wk_ref[:, :],
                                       preferred_element_type=jnp.float32)))
        commsc_ref[OWN, K_, 0, :] = k_sc[0, :]
        commk_ref[OWN, :, :, :] = k_q.reshape(B, SQ_LOCAL, DM)

        def copy(ref, src_slot, dst_slot, sem, dev):
            return pltpu.make_async_remote_copy(
                src_ref=ref.at[src_slot],
                dst_ref=ref.at[dst_slot],
                send_sem=send_sems.at[sem],
                recv_sem=recv_sems.at[sem],
                device_id=(dev,),
                device_id_type=pl.DeviceIdType.MESH,
            )

        barrier_sem = pltpu.get_barrier_semaphore()
        for nbr in (left, right):
            pl.semaphore_signal(
                barrier_sem, inc=1,
                device_id=(nbr,), device_id_type=pl.DeviceIdType.MESH,
            )
        pl.semaphore_wait(barrier_sem, 2)

        rdma_skr = copy(commsc_ref.at[:, K_], OWN, L, 6, right)
        rdma_skl = copy(commsc_ref.at[:, K_], OWN, R, 7, left)
        rdma_skr.start()
        rdma_skl.start()
        rdma_kr = copy(commk_ref, OWN, L, 0, right)
        rdma_kl = copy(commk_ref, OWN, R, 1, left)
        rdma_kr.start()
        rdma_kl.start()

        v_q, v_sc = quant(jnp.dot(x2, wv_ref[:, :],
                                  preferred_element_type=jnp.float32))
        commsc_ref[OWN, V_, 0, :] = v_sc[0, :]
        commv_ref[OWN, :, :, :] = v_q.reshape(B, SQ_LOCAL, DM)

        rdma_svr = copy(commsc_ref.at[:, V_], OWN, L, 8, right)
        rdma_svl = copy(commsc_ref.at[:, V_], OWN, R, 9, left)
        rdma_svr.start()
        rdma_svl.start()
        rdma_vr = copy(commv_ref, OWN, L, 2, right)
        rdma_vl = copy(commv_ref, OWN, R, 3, left)
        rdma_vr.start()
        rdma_vl.start()

        q_rope = rope(jnp.dot(x2, wq_ref[:, :],
                              preferred_element_type=jnp.float32)).astype(BF16)
        qs = [[q_rope[b * SQ_LOCAL:(b + 1) * SQ_LOCAL, hh * DH:(hh + 1) * DH]
               for hh in range(HQ)] for b in range(B)]

        state = {}

        def flash(slots):
            for b in range(B):
                k_all = jnp.concatenate(
                    [commk_ref[s, b, :, :].astype(BF16)
                     * commsc_ref[s, K_, 0, :].astype(BF16)[None, :]
                     for s in slots], axis=0)
                v_all = jnp.concatenate(
                    [commv_ref[s, b, :, :].astype(BF16)
                     * commsc_ref[s, V_, 0, :].astype(BF16)[None, :]
                     for s in slots], axis=0)
                for hh in range(HQ):
                    sl = slice(hh * DH, (hh + 1) * DH)
                    kh = k_all[:, sl]
                    vh = v_all[:, sl]
                    s_ = lax.dot_general(
                        qs[b][hh], kh, (((1,), (1,)), ((), ())),
                        preferred_element_type=jnp.float32,
                    ) * SCALE
                    m_c = jnp.max(s_, axis=1, keepdims=True)
                    if (b, hh) not in state:
                        p = jnp.exp(s_ - m_c)
                        acc = jnp.dot(p.astype(BF16), vh,
                                      preferred_element_type=jnp.float32)
                        state[(b, hh)] = (m_c, jnp.sum(p, axis=1, keepdims=True), acc)
                    else:
                        m, l, acc = state[(b, hh)]
                        m_new = jnp.maximum(m, m_c)
                        alpha = jnp.exp(m - m_new)
                        p = jnp.exp(s_ - m_new)
                        l = l * alpha + jnp.sum(p, axis=1, keepdims=True)
                        acc = acc * alpha + jnp.dot(
                            p.astype(BF16), vh, preferred_element_type=jnp.float32)
                        state[(b, hh)] = (m_new, l, acc)

        flash([OWN])

        rdma_skr.wait_recv()
        rdma_kr.wait_recv()
        rdma_fsk = copy(commsc_ref.at[:, K_], L, OPP, 10, right)
        rdma_fsk.start()
        rdma_fk = copy(commk_ref, L, OPP, 4, right)
        rdma_fk.start()

        rdma_svl.wait_recv()
        rdma_vl.wait_recv()
        rdma_fsv = copy(commsc_ref.at[:, V_], R, OPP, 11, left)
        rdma_fsv.start()
        rdma_fv = copy(commv_ref, R, OPP, 5, left)
        rdma_fv.start()

        rdma_skl.wait_recv()
        rdma_kl.wait_recv()
        rdma_svr.wait_recv()
        rdma_vr.wait_recv()
        flash([L, R])

        rdma_fsk.wait_recv()
        rdma_fk.wait_recv()
        rdma_fsv.wait_recv()
        rdma_fv.wait_recv()
        flash([OPP])

        ctx = jnp.concatenate(
            [jnp.concatenate(
                [state[(b, hh)][2] / state[(b, hh)][1] for hh in range(HQ)],
                axis=1)
             for b in range(B)], axis=0).astype(BF16)
        o2 = jnp.dot(ctx, wo_ref[:, :], preferred_element_type=jnp.float32)
        out_ref[:, :, :] = o2.astype(BF16).reshape(B, SQ_LOCAL, D)

        for r in (rdma_kr, rdma_kl, rdma_vr, rdma_vl, rdma_fk, rdma_fv,
                  rdma_skr, rdma_skl, rdma_svr, rdma_svl, rdma_fsk, rdma_fsv):
            r.wait_send()

    return pl.pallas_call(
        body,
        out_shape=jax.ShapeDtypeStruct((B, SQ_LOCAL, D), BF16),
        in_specs=[pl.BlockSpec(memory_space=pltpu.VMEM)] * 7,
        out_specs=pl.BlockSpec(memory_space=pltpu.VMEM),
        scratch_shapes=[
            pltpu.VMEM((N_DEV, B, SQ_LOCAL, DM), INT8),
            pltpu.VMEM((N_DEV, B, SQ_LOCAL, DM), INT8),
            pltpu.VMEM((N_DEV, 2, 8, DM), jnp.float32),
            pltpu.SemaphoreType.DMA((12,)),
            pltpu.SemaphoreType.DMA((12,)),
        ],
        compiler_params=pltpu.CompilerParams(collective_id=0),
    )(x.astype(BF16), Wq.astype(BF16), Wk.astype(BF16),
      Wv.astype(BF16), Wo.astype(BF16), cos_l, sin_l)


# device time: 19040 ns/iter; 1.0279x vs baseline; 1.0279x over previous
import numpy as np
import jax
import jax.numpy as jnp
from jax import lax
from jax.experimental import pallas as pl
from jax.experimental.pallas import tpu as pltpu

N_DEV = 4
B = 2
SQ_LOCAL = 256
SQ = SQ_LOCAL * N_DEV
D = 768
HQ = 4
DH = 64
DM = HQ * DH
SCALE = 1.0 / np.sqrt(DH)

BF16 = jnp.bfloat16
INT8 = jnp.int8

OWN, L, R, OPP = 0, 1, 2, 3
K_, V_ = 0, 1

_TABS = None


def _rope_tables():
    global _TABS
    if _TABS is None:
        inv = 1.0 / (10000.0 ** (np.arange(0, DH, 2) / DH))
        pos = np.arange(SQ)[:, None] * inv[None, :]
        cos = np.repeat(np.cos(pos), 2, axis=-1)
        sin = np.repeat(np.sin(pos), 2, axis=-1)
        _TABS = (jnp.asarray(np.tile(cos, (1, HQ)), dtype=jnp.bfloat16),
                 jnp.asarray(np.tile(sin, (1, HQ)), dtype=jnp.bfloat16))
    return _TABS


def kernel(x, Wq, Wk, Wv, Wo):
    cos_l, sin_l = _rope_tables()

    def body(xh_ref, wqh_ref, wkh_ref, wvh_ref, woh_ref, cosh_ref, sinh_ref,
             out_ref, x_vref, wq_ref, wk_ref, wv_ref, wo_ref, cos_ref, sin_ref,
             commk_ref, commv_ref, commsc_ref, send_sems, recv_sems, in_sems):
        my = lax.axis_index("i")

        stage = []
        for i, (h, v) in enumerate([
                (xh_ref, x_vref), (wkh_ref, wk_ref), (cosh_ref, cos_ref),
                (sinh_ref, sin_ref), (wvh_ref, wv_ref), (wqh_ref, wq_ref),
                (woh_ref, wo_ref)]):
            c = pltpu.make_async_copy(h, v, in_sems.at[i])
            c.start()
            stage.append(c)
        stage[0].wait()
        stage[1].wait()
        left = lax.rem(my + N_DEV - 1, N_DEV)
        right = lax.rem(my + 1, N_DEV)

        stage[2].wait()
        stage[3].wait()

        def make_rope(cos, sin):
            lane = lax.broadcasted_iota(jnp.int32, cos.shape, 1)
            even = (lane % 2) == 0

            def rope(t):
                r = jnp.where(even, -pltpu.roll(t, DM - 1, 1),
                              pltpu.roll(t, 1, 1))
                return t * cos + r * sin
            return rope

        cos_1 = cos_ref[pl.ds(my * SQ_LOCAL, SQ_LOCAL), :].astype(jnp.float32)
        sin_1 = sin_ref[pl.ds(my * SQ_LOCAL, SQ_LOCAL), :].astype(jnp.float32)
        rope_1 = make_rope(cos_1, sin_1)
        rope_2 = make_rope(jnp.concatenate([cos_1, cos_1], axis=0),
                           jnp.concatenate([sin_1, sin_1], axis=0))

        def quant(t):
            cmax = jnp.maximum(
                jnp.max(jnp.abs(t), axis=0, keepdims=True), 1e-20)
            q = jnp.clip(jnp.round(t * (127.0 / cmax)), -127.0, 127.0)
            return q.astype(INT8), cmax * (1.0 / 127.0)

        x2 = x_vref[:, :, :].reshape(B * SQ_LOCAL, D)

        kq0, ksc0 = quant(rope_1(jnp.dot(x2[0:SQ_LOCAL, :], wk_ref[:, :],
                                         preferred_element_type=jnp.float32)))
        commsc_ref[OWN, K_, 0, :] = ksc0[0, :]
        commk_ref[OWN, 0, :, :] = kq0

        def copy(ref, src_slot, dst_slot, sem, dev):
            return pltpu.make_async_remote_copy(
                src_ref=ref.at[src_slot],
                dst_ref=ref.at[dst_slot],
                send_sem=send_sems.at[sem],
                recv_sem=recv_sems.at[sem],
                device_id=(dev,),
                device_id_type=pl.DeviceIdType.MESH,
            )

        barrier_sem = pltpu.get_barrier_semaphore()
        for nbr in (left, right):
            pl.semaphore_signal(
                barrier_sem, inc=1,
                device_id=(nbr,), device_id_type=pl.DeviceIdType.MESH,
            )
        pl.semaphore_wait(barrier_sem, 2)

        rdma_skr0 = copy(commsc_ref.at[:, K_, pl.ds(0, 1)], OWN, L, 6, right)
        rdma_skl0 = copy(commsc_ref.at[:, K_, pl.ds(0, 1)], OWN, R, 7, left)
        rdma_skr0.start()
        rdma_skl0.start()
        rdma_kr0 = copy(commk_ref.at[:, 0], OWN, L, 0, right)
        rdma_kl0 = copy(commk_ref.at[:, 0], OWN, R, 1, left)
        rdma_kr0.start()
        rdma_kl0.start()

        kq1, ksc1 = quant(rope_1(jnp.dot(x2[SQ_LOCAL:, :], wk_ref[:, :],
                                         preferred_element_type=jnp.float32)))
        commsc_ref[OWN, K_, 1, :] = ksc1[0, :]
        commk_ref[OWN, 1, :, :] = kq1
        rdma_skr1 = copy(commsc_ref.at[:, K_, pl.ds(1, 1)], OWN, L, 12, right)
        rdma_skl1 = copy(commsc_ref.at[:, K_, pl.ds(1, 1)], OWN, R, 13, left)
        rdma_skr1.start()
        rdma_skl1.start()
        rdma_kr1 = copy(commk_ref.at[:, 1], OWN, L, 14, right)
        rdma_kl1 = copy(commk_ref.at[:, 1], OWN, R, 15, left)
        rdma_kr1.start()
        rdma_kl1.start()

        stage[4].wait()
        v_q, v_sc = quant(jnp.dot(x2, wv_ref[:, :],
                                  preferred_element_type=jnp.float32))
        commsc_ref[OWN, V_, 0, :] = v_sc[0, :]
        commv_ref[OWN, :, :, :] = v_q.reshape(B, SQ_LOCAL, DM)

        rdma_svr = copy(commsc_ref.at[:, V_], OWN, L, 8, right)
        rdma_svl = copy(commsc_ref.at[:, V_], OWN, R, 9, left)
        rdma_svr.start()
        rdma_svl.start()
        rdma_vr = copy(commv_ref, OWN, L, 2, right)
        rdma_vl = copy(commv_ref, OWN, R, 3, left)
        rdma_vr.start()
        rdma_vl.start()

        stage[5].wait()
        q_rope = rope_2(jnp.dot(x2, wq_ref[:, :],
                                preferred_element_type=jnp.float32)).astype(BF16)
        qs = [[q_rope[b * SQ_LOCAL:(b + 1) * SQ_LOCAL, hh * DH:(hh + 1) * DH]
               for hh in range(HQ)] for b in range(B)]

        state = {}

        def flash(slots):
            for b in range(B):
                k_all = jnp.concatenate(
                    [commk_ref[s, b, :, :].astype(BF16)
                     * commsc_ref[s, K_, b, :].astype(BF16)[None, :]
                     for s in slots], axis=0)
                v_all = jnp.concatenate(
                    [commv_ref[s, b, :, :].astype(BF16)
                     * commsc_ref[s, V_, 0, :].astype(BF16)[None, :]
                     for s in slots], axis=0)
                for hh in range(HQ):
                    sl = slice(hh * DH, (hh + 1) * DH)
                    kh = k_all[:, sl]
                    vh = v_all[:, sl]
                    s_ = lax.dot_general(
                        qs[b][hh], kh, (((1,), (1,)), ((), ())),
                        preferred_element_type=jnp.float32,
                    ) * SCALE
                    m_c = jnp.max(s_, axis=1, keepdims=True)
                    if (b, hh) not in state:
                        p = jnp.exp(s_ - m_c)
                        acc = jnp.dot(p.astype(BF16), vh,
                                      preferred_element_type=jnp.float32)
                        state[(b, hh)] = (m_c, jnp.sum(p, axis=1, keepdims=True), acc)
                    else:
                        m, l, acc = state[(b, hh)]
                        m_new = jnp.maximum(m, m_c)
                        alpha = jnp.exp(m - m_new)
                        p = jnp.exp(s_ - m_new)
                        l = l * alpha + jnp.sum(p, axis=1, keepdims=True)
                        acc = acc * alpha + jnp.dot(
                            p.astype(BF16), vh, preferred_element_type=jnp.float32)
                        state[(b, hh)] = (m_new, l, acc)

        flash([OWN])

        rdma_skr0.wait_recv()
        rdma_skr1.wait_recv()
        rdma_kr0.wait_recv()
        rdma_kr1.wait_recv()
        rdma_fsk = copy(commsc_ref.at[:, K_], L, OPP, 10, right)
        rdma_fsk.start()
        rdma_fk = copy(commk_ref, L, OPP, 4, right)
        rdma_fk.start()

        rdma_svl.wait_recv()
        rdma_vl.wait_recv()
        rdma_fsv = copy(commsc_ref.at[:, V_], R, OPP, 11, left)
        rdma_fsv.start()
        rdma_fv = copy(commv_ref, R, OPP, 5, left)
        rdma_fv.start()

        rdma_skl0.wait_recv()
        rdma_skl1.wait_recv()
        rdma_kl0.wait_recv()
        rdma_kl1.wait_recv()
        rdma_svr.wait_recv()
        rdma_vr.wait_recv()
        flash([L, R])

        rdma_fsk.wait_recv()
        rdma_fk.wait_recv()
        rdma_fsv.wait_recv()
        rdma_fv.wait_recv()
        flash([OPP])

        ctx = jnp.concatenate(
            [jnp.concatenate(
                [state[(b, hh)][2] / state[(b, hh)][1] for hh in range(HQ)],
                axis=1)
             for b in range(B)], axis=0).astype(BF16)
        stage[6].wait()
        o2 = jnp.dot(ctx, wo_ref[:, :], preferred_element_type=jnp.float32)
        out_ref[:, :, :] = o2.astype(BF16).reshape(B, SQ_LOCAL, D)

        for r in (rdma_kr0, rdma_kl0, rdma_kr1, rdma_kl1, rdma_vr, rdma_vl,
                  rdma_fk, rdma_fv, rdma_skr0, rdma_skl0, rdma_skr1,
                  rdma_skl1, rdma_svr, rdma_svl, rdma_fsk, rdma_fsv):
            r.wait_send()

    return pl.pallas_call(
        body,
        out_shape=jax.ShapeDtypeStruct((B, SQ_LOCAL, D), BF16),
        in_specs=[pl.BlockSpec(memory_space=pl.ANY)] * 7,
        out_specs=pl.BlockSpec(memory_space=pltpu.VMEM),
        scratch_shapes=[
            pltpu.VMEM((B, SQ_LOCAL, D), BF16),
            pltpu.VMEM((D, DM), BF16),
            pltpu.VMEM((D, DM), BF16),
            pltpu.VMEM((D, DM), BF16),
            pltpu.VMEM((DM, D), BF16),
            pltpu.VMEM((SQ, DM), BF16),
            pltpu.VMEM((SQ, DM), BF16),
            pltpu.VMEM((N_DEV, B, SQ_LOCAL, DM), INT8),
            pltpu.VMEM((N_DEV, B, SQ_LOCAL, DM), INT8),
            pltpu.VMEM((N_DEV, 2, 8, DM), jnp.float32),
            pltpu.SemaphoreType.DMA((16,)),
            pltpu.SemaphoreType.DMA((16,)),
            pltpu.SemaphoreType.DMA((7,)),
        ],
        compiler_params=pltpu.CompilerParams(collective_id=0),
    )(x.astype(BF16), Wq.astype(BF16), Wk.astype(BF16),
      Wv.astype(BF16), Wo.astype(BF16), cos_l, sin_l)
